# baseline (speedup 1.0000x reference)
import jax
import jax.numpy as jnp
from jax.experimental import pallas as pl
from jax.experimental.pallas import tpu as pltpu

_M = 8192
_R, _C = 64, 128
_G = 4
_BR = _R // _G


def _fold8(x, op2):
    r = x.shape[0]
    while r > 8:
        r //= 2
        x = op2(x[0:r], x[r:2 * r])
    return x


def _quantize_body(inp_ref, cb_ref, out_ref, alo_ref, ahi_ref):
    i = pl.program_id(0)
    v = inp_ref[0, 0]
    cb = cb_ref[...]

    ninf = jnp.float32(-jnp.inf)
    pinf = jnp.float32(jnp.inf)
    lt = cb < v
    part_lo = _fold8(jnp.where(lt, cb, ninf), jnp.maximum)
    part_hi = _fold8(jnp.where(lt, pinf, cb), jnp.minimum)

    @pl.when(i == 0)
    def _():
        alo_ref[...] = part_lo
        ahi_ref[...] = part_hi

    @pl.when(i > 0)
    def _():
        alo_ref[...] = jnp.maximum(alo_ref[...], part_lo)
        ahi_ref[...] = jnp.minimum(ahi_ref[...], part_hi)

    @pl.when(i == _G - 1)
    def _():
        def red11(x, op):
            return op(op(x, axis=1, keepdims=True), axis=0, keepdims=True)

        g_lo = red11(alo_ref[...], jnp.max)
        g_hi = red11(ahi_ref[...], jnp.min)
        g0 = cb_ref[0:1, 0:1]

        res = jnp.where(v <= (g_hi - g_lo) / 2, g_lo, g_hi)
        res = jnp.where(g_hi == v, g0, res)
        res = jnp.where(g_lo == ninf, g0, res)
        out_ref[...] = res


@jax.jit
def _quantize(inp_s, cb2d):
    return pl.pallas_call(
        _quantize_body,
        grid=(_G,),
        in_specs=[
            pl.BlockSpec(memory_space=pltpu.SMEM),
            pl.BlockSpec((_BR, _C), lambda i: (_G - 1 - i, 0)),
        ],
        out_specs=pl.BlockSpec((1, 1), lambda i: (0, 0)),
        out_shape=jax.ShapeDtypeStruct((1, 1), jnp.float32),
        scratch_shapes=[
            pltpu.VMEM((8, _C), jnp.float32),
            pltpu.VMEM((8, _C), jnp.float32),
        ],
    )(inp_s, cb2d)


def kernel(input, codebook):
    cb2d = codebook.reshape(_R, _C)
    return _quantize(input.reshape(1, 1), cb2d).reshape(1)


# speedup vs baseline: 1.3680x; 1.3680x over previous
import jax
import jax.numpy as jnp
from jax.experimental import pallas as pl
from jax.experimental.pallas import tpu as pltpu

_M = 8192
_R, _C = 64, 128


def _red11(x, op2, opred):
    x = op2(x[0:32], x[32:64])
    x = op2(x[0:16], x[16:32])
    x = op2(x[0:8], x[8:16])
    return opred(opred(x, axis=1, keepdims=True), axis=0, keepdims=True)


def _quantize_body(inp_ref, cb_ref, out_ref):
    v = inp_ref[0, 0]
    cb = cb_ref[...]

    ninf = jnp.float32(-jnp.inf)
    pinf = jnp.float32(jnp.inf)
    lt = cb < v
    g_lo = _red11(jnp.where(lt, cb, ninf), jnp.maximum, jnp.max)
    g_hi = _red11(jnp.where(lt, pinf, cb), jnp.minimum, jnp.min)
    g0 = cb[0:1, 0:1]

    res = jnp.where(v <= (g_hi - g_lo) / 2, g_lo, g_hi)
    res = jnp.where(g_hi == v, g0, res)
    res = jnp.where(g_lo == ninf, g0, res)
    out_ref[...] = res


@jax.jit
def _quantize(inp11, cb2d):
    return pl.pallas_call(
        _quantize_body,
        in_specs=[
            pl.BlockSpec(memory_space=pltpu.SMEM),
            pl.BlockSpec(memory_space=pltpu.VMEM),
        ],
        out_specs=pl.BlockSpec(memory_space=pltpu.VMEM),
        out_shape=jax.ShapeDtypeStruct((1, 1), jnp.float32),
    )(inp11, cb2d)


def kernel(input, codebook):
    inp11 = input.reshape(1, 1)
    cb2d = codebook.reshape(_R, _C)
    return _quantize(inp11, cb2d).reshape(1)
